# baseline (speedup 1.0000x reference)
import functools

import jax
import jax.numpy as jnp
from jax import lax
from jax.experimental import pallas as pl
from jax.experimental.pallas import tpu as pltpu
from jax.experimental.pallas import tpu_sc as plsc

N = 10000
D = 128
H = 128
OUT = 64
B = 512

NCORES = 2
NSUB = 16
NTILES = NCORES * NSUB
CHUNK = 128
NACC = 10112
RPT = NACC // NSUB
DUMMY = N
CW = 16


def _edge_body(xl_h, idx_h, znh_h, ones_h, agg_o, cnt_o,
               idx_v, rows_v, agg_s, sem0, sem1, want_cnt):
    cid = lax.axis_index("c")
    sid = lax.axis_index("s")
    wid = cid * NSUB + sid
    r0 = sid * RPT
    L = idx_h.shape[1] - 1
    pltpu.sync_copy(znh_h.at[pl.ds(r0, RPT)], agg_s.at[pl.ds(r0, RPT)])
    plsc.subcore_barrier()

    pltpu.sync_copy(idx_h.at[wid, 0], idx_v.at[0])
    pltpu.async_copy(xl_h.at[idx_v.at[0, 0]], rows_v.at[0], sem0)

    def pair(i, carry):
        j = 2 * i
        pltpu.sync_copy(idx_h.at[wid, j + 1], idx_v.at[1])
        pltpu.async_copy(xl_h.at[idx_v.at[1, 0]], rows_v.at[1], sem1)
        pltpu.make_async_copy(xl_h.at[pl.ds(0, CHUNK)], rows_v.at[0], sem0).wait()
        pltpu.sync_copy(rows_v.at[0], agg_s.at[idx_v.at[0, 1]], add=True)
        pltpu.sync_copy(idx_h.at[wid, j + 2], idx_v.at[0])
        pltpu.async_copy(xl_h.at[idx_v.at[0, 0]], rows_v.at[0], sem0)
        pltpu.make_async_copy(xl_h.at[pl.ds(0, CHUNK)], rows_v.at[1], sem1).wait()
        pltpu.sync_copy(rows_v.at[1], agg_s.at[idx_v.at[1, 1]], add=True)
        return carry

    lax.fori_loop(0, L // 2, pair, 0)
    pltpu.make_async_copy(xl_h.at[pl.ds(0, CHUNK)], rows_v.at[0], sem0).wait()
    plsc.subcore_barrier()
    pltpu.sync_copy(agg_s.at[pl.ds(r0, RPT)], agg_o.at[cid, pl.ds(r0, RPT)])

    if want_cnt:
        plsc.subcore_barrier()
        pltpu.sync_copy(znh_h.at[pl.ds(r0, RPT)], agg_s.at[pl.ds(r0, RPT)])
        pltpu.sync_copy(ones_h, rows_v.at[0])
        plsc.subcore_barrier()
        pltpu.sync_copy(idx_h.at[wid, 0], idx_v.at[0])

        def cpair(i, carry):
            j = 2 * i
            h1 = pltpu.async_copy(idx_h.at[wid, j + 1], idx_v.at[1], sem1)
            pltpu.sync_copy(rows_v.at[0], agg_s.at[idx_v.at[0, 1]], add=True)
            h1.wait()
            h0 = pltpu.async_copy(idx_h.at[wid, j + 2], idx_v.at[0], sem0)
            pltpu.sync_copy(rows_v.at[0], agg_s.at[idx_v.at[1, 1]], add=True)
            h0.wait()
            return carry

        lax.fori_loop(0, L // 2, cpair, 0)
        plsc.subcore_barrier()
        pltpu.sync_copy(agg_s.at[pl.ds(r0, RPT)], cnt_o.at[cid, pl.ds(r0, RPT)])


def _edge_pass0(xl, idx3, znh, onesH):
    mesh = plsc.VectorSubcoreMesh(core_axis_name="c", subcore_axis_name="s")
    k = pl.kernel(
        functools.partial(_edge_body, want_cnt=True),
        out_type=[
            jax.ShapeDtypeStruct((NCORES, NACC, H), jnp.float32),
            jax.ShapeDtypeStruct((NCORES, NACC, H), jnp.float32),
        ],
        mesh=mesh,
        scratch_types=[
            pltpu.VMEM((2, 2, CHUNK), jnp.int32),
            pltpu.VMEM((2, CHUNK, H), jnp.float32),
            pltpu.VMEM_SHARED((NACC, H), jnp.float32),
            pltpu.SemaphoreType.DMA,
            pltpu.SemaphoreType.DMA,
        ],
    )
    return k(xl, idx3, znh, onesH)


def _edge_body1(xl_h, idx_h, znh_h, ones_h, agg_o, idx_v, rows_v, agg_s,
                sem0, sem1):
    return _edge_body(xl_h, idx_h, znh_h, ones_h, agg_o, None,
                      idx_v, rows_v, agg_s, sem0, sem1, want_cnt=False)


def _edge_pass1(xl, idx3, znh, onesH):
    mesh = plsc.VectorSubcoreMesh(core_axis_name="c", subcore_axis_name="s")
    k = pl.kernel(
        _edge_body1,
        out_type=jax.ShapeDtypeStruct((NCORES, NACC, H), jnp.float32),
        mesh=mesh,
        scratch_types=[
            pltpu.VMEM((2, 2, CHUNK), jnp.int32),
            pltpu.VMEM((2, CHUNK, H), jnp.float32),
            pltpu.VMEM_SHARED((NACC, H), jnp.float32),
            pltpu.SemaphoreType.DMA,
            pltpu.SemaphoreType.DMA,
        ],
    )
    return k(xl, idx3, znh, onesH)


RB = 1000
RBF = 400


def _pre_body(x_ref, wl_ref, wr_ref, b_ref, xl_ref, xr_ref):
    x = x_ref[...]
    xl_ref[...] = jnp.dot(x, wl_ref[...], preferred_element_type=jnp.float32)
    xr_ref[...] = jnp.dot(x, wr_ref[...], preferred_element_type=jnp.float32) + b_ref[...]


def _pre(x, wl, wr, b):
    return pl.pallas_call(
        _pre_body,
        grid=(N // RB,),
        in_specs=[
            pl.BlockSpec((RB, D), lambda i: (i, 0)),
            pl.BlockSpec((D, H), lambda i: (0, 0)),
            pl.BlockSpec((D, H), lambda i: (0, 0)),
            pl.BlockSpec((1, H), lambda i: (0, 0)),
        ],
        out_specs=[
            pl.BlockSpec((RB, H), lambda i: (i, 0)),
            pl.BlockSpec((RB, H), lambda i: (i, 0)),
        ],
        out_shape=[
            jax.ShapeDtypeStruct((N, H), jnp.float32),
            jax.ShapeDtypeStruct((N, H), jnp.float32),
        ],
    )(x, wl, wr, b)


def _mid_body(a0_ref, a1_ref, c0_ref, c1_ref, xr_ref, wl_ref, wr_ref, b_ref,
              xl1_ref, xr1_ref):
    cnt = jnp.maximum(c0_ref[...][:, 0:1] + c1_ref[...][:, 0:1], 1.0)
    h = jnp.maximum((a0_ref[...] + a1_ref[...]) / cnt + xr_ref[...], 0.0)
    xl1_ref[...] = jnp.dot(h, wl_ref[...], preferred_element_type=jnp.float32)
    xr1_ref[...] = jnp.dot(h, wr_ref[...], preferred_element_type=jnp.float32) + b_ref[...]


def _mid(a0, a1, c0, c1, xr, wl, wr, b):
    return pl.pallas_call(
        _mid_body,
        grid=(N // RB,),
        in_specs=[
            pl.BlockSpec((RB, H), lambda i: (i, 0)),
            pl.BlockSpec((RB, H), lambda i: (i, 0)),
            pl.BlockSpec((RB, H), lambda i: (i, 0)),
            pl.BlockSpec((RB, H), lambda i: (i, 0)),
            pl.BlockSpec((RB, H), lambda i: (i, 0)),
            pl.BlockSpec((H, H), lambda i: (0, 0)),
            pl.BlockSpec((H, H), lambda i: (0, 0)),
            pl.BlockSpec((1, H), lambda i: (0, 0)),
        ],
        out_specs=[
            pl.BlockSpec((RB, H), lambda i: (i, 0)),
            pl.BlockSpec((RB, H), lambda i: (i, 0)),
        ],
        out_shape=[
            jax.ShapeDtypeStruct((N, H), jnp.float32),
            jax.ShapeDtypeStruct((N, H), jnp.float32),
        ],
    )(a0, a1, c0, c1, xr, wl, wr, b)


def _fin_body(a0_ref, a1_ref, c0_ref, c1_ref, xr_ref, batch_ref, root_ref,
              wrl_ref, wpl_ref, blin_ref, out_ref, pooled_s, roots_s, csum_s):
    i = pl.program_id(0)

    @pl.when(i == 0)
    def _():
        pooled_s[...] = jnp.zeros_like(pooled_s)
        roots_s[...] = jnp.zeros_like(roots_s)
        csum_s[...] = jnp.zeros_like(csum_s)

    cnt = jnp.maximum(c0_ref[...][:, 0:1] + c1_ref[...][:, 0:1], 1.0)
    h = jnp.maximum((a0_ref[...] + a1_ref[...]) / cnt + xr_ref[...], 0.0)
    bv = batch_ref[...][:, 0]
    gids = lax.broadcasted_iota(jnp.int32, (B, RBF), 0)
    msel = (bv[None, :] == gids).astype(jnp.float32)
    rowids = i * RBF + lax.broadcasted_iota(jnp.int32, (B, RBF), 1)
    rsel = (root_ref[...][:, 0:1] == rowids).astype(jnp.float32)
    pooled_s[...] += jnp.dot(msel, h, preferred_element_type=jnp.float32)
    roots_s[...] += jnp.dot(rsel, h, preferred_element_type=jnp.float32)
    csum_s[...] = csum_s[...] + jnp.sum(msel, axis=1, keepdims=True)

    @pl.when(i == pl.num_programs(0) - 1)
    def _():
        pooled = pooled_s[...] / jnp.maximum(csum_s[...], 1.0)
        logits = (jnp.dot(roots_s[...], wrl_ref[...], preferred_element_type=jnp.float32)
                  + jnp.dot(pooled, wpl_ref[...], preferred_element_type=jnp.float32)
                  + blin_ref[...])
        m = jnp.max(logits, axis=-1, keepdims=True)
        e = logits - m
        lse = jnp.log(jnp.sum(jnp.exp(e), axis=-1, keepdims=True))
        out_ref[...] = e - lse


def _fin(a0, a1, c0, c1, xr, batch8, root8, wrl, wpl, blin):
    return pl.pallas_call(
        _fin_body,
        grid=(N // RBF,),
        in_specs=[
            pl.BlockSpec((RBF, H), lambda i: (i, 0)),
            pl.BlockSpec((RBF, H), lambda i: (i, 0)),
            pl.BlockSpec((RBF, H), lambda i: (i, 0)),
            pl.BlockSpec((RBF, H), lambda i: (i, 0)),
            pl.BlockSpec((RBF, H), lambda i: (i, 0)),
            pl.BlockSpec((RBF, 8), lambda i: (i, 0)),
            pl.BlockSpec((B, 8), lambda i: (0, 0)),
            pl.BlockSpec((H, OUT), lambda i: (0, 0)),
            pl.BlockSpec((H, OUT), lambda i: (0, 0)),
            pl.BlockSpec((1, OUT), lambda i: (0, 0)),
        ],
        out_specs=pl.BlockSpec((B, OUT), lambda i: (0, 0)),
        out_shape=jax.ShapeDtypeStruct((B, OUT), jnp.float32),
        scratch_shapes=[
            pltpu.VMEM((B, H), jnp.float32),
            pltpu.VMEM((B, H), jnp.float32),
            pltpu.VMEM((B, H), jnp.float32),
        ],
    )(a0, a1, c0, c1, xr, batch8, root8, wrl, wpl, blin)


def kernel(x, edge_index, batch, root_n_id, Wl0, Wr0, b0, Wl1, Wr1, b1, Wlin, blin):
    E = edge_index.shape[1]
    ei = edge_index.astype(jnp.int32)
    cpt = -(-E // (NTILES * CHUNK))
    cpt = cpt + (cpt % 2)
    epad = NTILES * cpt * CHUNK
    src3 = jnp.concatenate(
        [ei[0], jnp.zeros((epad - E,), jnp.int32)]).reshape(NTILES, cpt, 1, CHUNK)
    dst3 = jnp.concatenate(
        [ei[1], jnp.full((epad - E,), DUMMY, jnp.int32)]).reshape(NTILES, cpt, 1, CHUNK)
    idx3 = jnp.concatenate([src3, dst3], axis=2)
    dummy = jnp.broadcast_to(
        jnp.array([0, DUMMY], jnp.int32)[None, None, :, None],
        (NTILES, 1, 2, CHUNK))
    idx3 = jnp.concatenate([idx3, dummy], axis=1)
    znh = jnp.zeros((NACC, H), jnp.float32)
    onesH = jnp.ones((CHUNK, H), jnp.float32)
    batch8 = jnp.broadcast_to(batch.astype(jnp.int32)[:, None], (N, 8))
    root8 = jnp.broadcast_to(root_n_id.astype(jnp.int32)[:, None], (B, 8))
    b0r = b0.reshape(1, H)
    b1r = b1.reshape(1, H)
    blinr = blin.reshape(1, OUT)
    wrl = Wlin[:H]
    wpl = Wlin[H:]

    xl0, xr0 = _pre(x, Wl0, Wr0, b0r)
    aggp0, cntp = _edge_pass0(xl0, idx3, znh, onesH)
    xl1, xr1 = _mid(aggp0[0, :N], aggp0[1, :N], cntp[0, :N], cntp[1, :N],
                    xr0, Wl1, Wr1, b1r)
    aggp1 = _edge_pass1(xl1, idx3, znh, onesH)
    return _fin(aggp1[0, :N], aggp1[1, :N], cntp[0, :N], cntp[1, :N],
                xr1, batch8, root8, wrl, wpl, blinr)


# speedup vs baseline: 1.0558x; 1.0558x over previous
import functools

import jax
import jax.numpy as jnp
from jax import lax
from jax.experimental import pallas as pl
from jax.experimental.pallas import tpu as pltpu
from jax.experimental.pallas import tpu_sc as plsc

N = 10000
D = 128
H = 128
OUT = 64
B = 512

NCORES = 2
NSUB = 16
NTILES = NCORES * NSUB
CHUNK = 128
NACC = 10112
RPT = NACC // NSUB
DUMMY = N
CW = 16


def _edge_body(xl_h, idx_h, znh_h, ones_h, agg_o, cnt_o,
               idx_v, rows_v, agg_s, sem0, sem1, want_cnt):
    cid = lax.axis_index("c")
    sid = lax.axis_index("s")
    wid = cid * NSUB + sid
    r0 = sid * RPT
    L = idx_h.shape[1] - 1
    pltpu.sync_copy(znh_h.at[pl.ds(r0, RPT)], agg_s.at[pl.ds(r0, RPT)])
    plsc.subcore_barrier()

    def body(j, carry):
        pltpu.sync_copy(idx_h.at[wid, j], idx_v.at[0])
        pltpu.async_copy(xl_h.at[idx_v.at[0, 0]], rows_v.at[0], sem0)
        pltpu.make_async_copy(xl_h.at[pl.ds(0, CHUNK)], rows_v.at[0], sem0).wait()
        pltpu.sync_copy(rows_v.at[0], agg_s.at[idx_v.at[0, 1]], add=True)
        return carry

    lax.fori_loop(0, L, body, 0)
    plsc.subcore_barrier()
    pltpu.sync_copy(agg_s.at[pl.ds(r0, RPT)], agg_o.at[cid, pl.ds(r0, RPT)])

    if want_cnt:
        plsc.subcore_barrier()
        pltpu.sync_copy(znh_h.at[pl.ds(r0, RPT)], agg_s.at[pl.ds(r0, RPT)])
        pltpu.sync_copy(ones_h, rows_v.at[0])
        plsc.subcore_barrier()
        def cbody(j, carry):
            pltpu.sync_copy(idx_h.at[wid, j], idx_v.at[0])
            pltpu.sync_copy(rows_v.at[0], agg_s.at[idx_v.at[0, 1]], add=True)
            return carry

        lax.fori_loop(0, L, cbody, 0)
        plsc.subcore_barrier()
        pltpu.sync_copy(agg_s.at[pl.ds(r0, RPT)], cnt_o.at[cid, pl.ds(r0, RPT)])


def _edge_pass0(xl, idx3, znh, onesH):
    mesh = plsc.VectorSubcoreMesh(core_axis_name="c", subcore_axis_name="s")
    k = pl.kernel(
        functools.partial(_edge_body, want_cnt=True),
        out_type=[
            jax.ShapeDtypeStruct((NCORES, NACC, H), jnp.float32),
            jax.ShapeDtypeStruct((NCORES, NACC, H), jnp.float32),
        ],
        mesh=mesh,
        scratch_types=[
            pltpu.VMEM((2, 2, CHUNK), jnp.int32),
            pltpu.VMEM((2, CHUNK, H), jnp.float32),
            pltpu.VMEM_SHARED((NACC, H), jnp.float32),
            pltpu.SemaphoreType.DMA,
            pltpu.SemaphoreType.DMA,
        ],
    )
    return k(xl, idx3, znh, onesH)


def _edge_body1(xl_h, idx_h, znh_h, ones_h, agg_o, idx_v, rows_v, agg_s,
                sem0, sem1):
    return _edge_body(xl_h, idx_h, znh_h, ones_h, agg_o, None,
                      idx_v, rows_v, agg_s, sem0, sem1, want_cnt=False)


def _edge_pass1(xl, idx3, znh, onesH):
    mesh = plsc.VectorSubcoreMesh(core_axis_name="c", subcore_axis_name="s")
    k = pl.kernel(
        _edge_body1,
        out_type=jax.ShapeDtypeStruct((NCORES, NACC, H), jnp.float32),
        mesh=mesh,
        scratch_types=[
            pltpu.VMEM((2, 2, CHUNK), jnp.int32),
            pltpu.VMEM((2, CHUNK, H), jnp.float32),
            pltpu.VMEM_SHARED((NACC, H), jnp.float32),
            pltpu.SemaphoreType.DMA,
            pltpu.SemaphoreType.DMA,
        ],
    )
    return k(xl, idx3, znh, onesH)


RB = 1000
RBF = 400


def _pre_body(x_ref, wl_ref, wr_ref, b_ref, xl_ref, xr_ref):
    x = x_ref[...]
    xl_ref[...] = jnp.dot(x, wl_ref[...], preferred_element_type=jnp.float32)
    xr_ref[...] = jnp.dot(x, wr_ref[...], preferred_element_type=jnp.float32) + b_ref[...]


def _pre(x, wl, wr, b):
    return pl.pallas_call(
        _pre_body,
        grid=(N // RB,),
        in_specs=[
            pl.BlockSpec((RB, D), lambda i: (i, 0)),
            pl.BlockSpec((D, H), lambda i: (0, 0)),
            pl.BlockSpec((D, H), lambda i: (0, 0)),
            pl.BlockSpec((1, H), lambda i: (0, 0)),
        ],
        out_specs=[
            pl.BlockSpec((RB, H), lambda i: (i, 0)),
            pl.BlockSpec((RB, H), lambda i: (i, 0)),
        ],
        out_shape=[
            jax.ShapeDtypeStruct((N, H), jnp.float32),
            jax.ShapeDtypeStruct((N, H), jnp.float32),
        ],
    )(x, wl, wr, b)


def _mid_body(a0_ref, a1_ref, c0_ref, c1_ref, xr_ref, wl_ref, wr_ref, b_ref,
              xl1_ref, xr1_ref):
    cnt = jnp.maximum(c0_ref[...][:, 0:1] + c1_ref[...][:, 0:1], 1.0)
    h = jnp.maximum((a0_ref[...] + a1_ref[...]) / cnt + xr_ref[...], 0.0)
    xl1_ref[...] = jnp.dot(h, wl_ref[...], preferred_element_type=jnp.float32)
    xr1_ref[...] = jnp.dot(h, wr_ref[...], preferred_element_type=jnp.float32) + b_ref[...]


def _mid(a0, a1, c0, c1, xr, wl, wr, b):
    return pl.pallas_call(
        _mid_body,
        grid=(N // RB,),
        in_specs=[
            pl.BlockSpec((RB, H), lambda i: (i, 0)),
            pl.BlockSpec((RB, H), lambda i: (i, 0)),
            pl.BlockSpec((RB, H), lambda i: (i, 0)),
            pl.BlockSpec((RB, H), lambda i: (i, 0)),
            pl.BlockSpec((RB, H), lambda i: (i, 0)),
            pl.BlockSpec((H, H), lambda i: (0, 0)),
            pl.BlockSpec((H, H), lambda i: (0, 0)),
            pl.BlockSpec((1, H), lambda i: (0, 0)),
        ],
        out_specs=[
            pl.BlockSpec((RB, H), lambda i: (i, 0)),
            pl.BlockSpec((RB, H), lambda i: (i, 0)),
        ],
        out_shape=[
            jax.ShapeDtypeStruct((N, H), jnp.float32),
            jax.ShapeDtypeStruct((N, H), jnp.float32),
        ],
    )(a0, a1, c0, c1, xr, wl, wr, b)


def _fin_body(a0_ref, a1_ref, c0_ref, c1_ref, xr_ref, batch_ref, root_ref,
              wrl_ref, wpl_ref, blin_ref, out_ref, pooled_s, roots_s, csum_s):
    i = pl.program_id(0)

    @pl.when(i == 0)
    def _():
        pooled_s[...] = jnp.zeros_like(pooled_s)
        roots_s[...] = jnp.zeros_like(roots_s)
        csum_s[...] = jnp.zeros_like(csum_s)

    cnt = jnp.maximum(c0_ref[...][:, 0:1] + c1_ref[...][:, 0:1], 1.0)
    h = jnp.maximum((a0_ref[...] + a1_ref[...]) / cnt + xr_ref[...], 0.0)
    bv = batch_ref[...][:, 0]
    gids = lax.broadcasted_iota(jnp.int32, (B, RBF), 0)
    msel = (bv[None, :] == gids).astype(jnp.float32)
    rowids = i * RBF + lax.broadcasted_iota(jnp.int32, (B, RBF), 1)
    rsel = (root_ref[...][:, 0:1] == rowids).astype(jnp.float32)
    pooled_s[...] += jnp.dot(msel, h, preferred_element_type=jnp.float32)
    roots_s[...] += jnp.dot(rsel, h, preferred_element_type=jnp.float32)
    csum_s[...] = csum_s[...] + jnp.sum(msel, axis=1, keepdims=True)

    @pl.when(i == pl.num_programs(0) - 1)
    def _():
        pooled = pooled_s[...] / jnp.maximum(csum_s[...], 1.0)
        logits = (jnp.dot(roots_s[...], wrl_ref[...], preferred_element_type=jnp.float32)
                  + jnp.dot(pooled, wpl_ref[...], preferred_element_type=jnp.float32)
                  + blin_ref[...])
        m = jnp.max(logits, axis=-1, keepdims=True)
        e = logits - m
        lse = jnp.log(jnp.sum(jnp.exp(e), axis=-1, keepdims=True))
        out_ref[...] = e - lse


def _fin(a0, a1, c0, c1, xr, batch8, root8, wrl, wpl, blin):
    return pl.pallas_call(
        _fin_body,
        grid=(N // RBF,),
        in_specs=[
            pl.BlockSpec((RBF, H), lambda i: (i, 0)),
            pl.BlockSpec((RBF, H), lambda i: (i, 0)),
            pl.BlockSpec((RBF, H), lambda i: (i, 0)),
            pl.BlockSpec((RBF, H), lambda i: (i, 0)),
            pl.BlockSpec((RBF, H), lambda i: (i, 0)),
            pl.BlockSpec((RBF, 8), lambda i: (i, 0)),
            pl.BlockSpec((B, 8), lambda i: (0, 0)),
            pl.BlockSpec((H, OUT), lambda i: (0, 0)),
            pl.BlockSpec((H, OUT), lambda i: (0, 0)),
            pl.BlockSpec((1, OUT), lambda i: (0, 0)),
        ],
        out_specs=pl.BlockSpec((B, OUT), lambda i: (0, 0)),
        out_shape=jax.ShapeDtypeStruct((B, OUT), jnp.float32),
        scratch_shapes=[
            pltpu.VMEM((B, H), jnp.float32),
            pltpu.VMEM((B, H), jnp.float32),
            pltpu.VMEM((B, H), jnp.float32),
        ],
    )(a0, a1, c0, c1, xr, batch8, root8, wrl, wpl, blin)


def kernel(x, edge_index, batch, root_n_id, Wl0, Wr0, b0, Wl1, Wr1, b1, Wlin, blin):
    E = edge_index.shape[1]
    ei = edge_index.astype(jnp.int32)
    cpt = -(-E // (NTILES * CHUNK))
    cpt = cpt + (cpt % 2)
    epad = NTILES * cpt * CHUNK
    src3 = jnp.concatenate(
        [ei[0], jnp.zeros((epad - E,), jnp.int32)]).reshape(NTILES, cpt, 1, CHUNK)
    dst3 = jnp.concatenate(
        [ei[1], jnp.full((epad - E,), DUMMY, jnp.int32)]).reshape(NTILES, cpt, 1, CHUNK)
    idx3 = jnp.concatenate([src3, dst3], axis=2)
    dummy = jnp.broadcast_to(
        jnp.array([0, DUMMY], jnp.int32)[None, None, :, None],
        (NTILES, 1, 2, CHUNK))
    idx3 = jnp.concatenate([idx3, dummy], axis=1)
    znh = jnp.zeros((NACC, H), jnp.float32)
    onesH = jnp.ones((CHUNK, H), jnp.float32)
    batch8 = jnp.broadcast_to(batch.astype(jnp.int32)[:, None], (N, 8))
    root8 = jnp.broadcast_to(root_n_id.astype(jnp.int32)[:, None], (B, 8))
    b0r = b0.reshape(1, H)
    b1r = b1.reshape(1, H)
    blinr = blin.reshape(1, OUT)
    wrl = Wlin[:H]
    wpl = Wlin[H:]

    xl0, xr0 = _pre(x, Wl0, Wr0, b0r)
    aggp0, cntp = _edge_pass0(xl0, idx3, znh, onesH)
    xl1, xr1 = _mid(aggp0[0, :N], aggp0[1, :N], cntp[0, :N], cntp[1, :N],
                    xr0, Wl1, Wr1, b1r)
    aggp1 = _edge_pass1(xl1, idx3, znh, onesH)
    return _fin(aggp1[0, :N], aggp1[1, :N], cntp[0, :N], cntp[1, :N],
                xr1, batch8, root8, wrl, wpl, blinr)


# speedup vs baseline: 1.3487x; 1.2775x over previous
import functools

import jax
import jax.numpy as jnp
from jax import lax
from jax.experimental import pallas as pl
from jax.experimental.pallas import tpu as pltpu
from jax.experimental.pallas import tpu_sc as plsc

N = 10000
D = 128
H = 128
OUT = 64
B = 512

NCORES = 2
NSUB = 16
NTILES = NCORES * NSUB
CHUNK = 128
NACC = 10112
RPT = NACC // NSUB
DUMMY = N
CW = 16


def _edge_body(xl_h, idx_h, znh_h, ones_h, agg_o, cnt_o,
               idx_v, rows_v, agg_s, sem, want_cnt):
    cid = lax.axis_index("c")
    sid = lax.axis_index("s")
    wid = cid * NSUB + sid
    r0 = sid * RPT
    L = idx_h.shape[1]
    pltpu.sync_copy(znh_h.at[pl.ds(r0, RPT)], agg_s.at[pl.ds(r0, RPT)])
    plsc.subcore_barrier()

    def body(j, carry):
        pltpu.sync_copy(idx_h.at[wid, j], idx_v)
        pltpu.async_copy(xl_h.at[idx_v.at[0]], rows_v, sem).wait()
        pltpu.sync_copy(rows_v, agg_s.at[idx_v.at[1]], add=True)
        return carry

    lax.fori_loop(0, L, body, 0)
    plsc.subcore_barrier()
    pltpu.sync_copy(agg_s.at[pl.ds(r0, RPT)], agg_o.at[cid, pl.ds(r0, RPT)])

    if want_cnt:
        plsc.subcore_barrier()
        pltpu.sync_copy(znh_h.at[pl.ds(r0, RPT)], agg_s.at[pl.ds(r0, RPT)])
        pltpu.sync_copy(ones_h, rows_v)
        plsc.subcore_barrier()

        def cbody(j, carry):
            pltpu.sync_copy(idx_h.at[wid, j], idx_v)
            pltpu.sync_copy(rows_v, agg_s.at[idx_v.at[1]], add=True)
            return carry

        lax.fori_loop(0, L, cbody, 0)
        plsc.subcore_barrier()
        pltpu.sync_copy(agg_s.at[pl.ds(r0, RPT)], cnt_o.at[cid, pl.ds(r0, RPT)])


def _edge_pass0(xl, idx3, znh, onesH):
    mesh = plsc.VectorSubcoreMesh(core_axis_name="c", subcore_axis_name="s")
    k = pl.kernel(
        functools.partial(_edge_body, want_cnt=True),
        out_type=[
            jax.ShapeDtypeStruct((NCORES, NACC, H), jnp.float32),
            jax.ShapeDtypeStruct((NCORES, NACC, H), jnp.float32),
        ],
        mesh=mesh,
        scratch_types=[
            pltpu.VMEM((2, CHUNK), jnp.int32),
            pltpu.VMEM((CHUNK, H), jnp.float32),
            pltpu.VMEM_SHARED((NACC, H), jnp.float32),
            pltpu.SemaphoreType.DMA,
        ],
    )
    return k(xl, idx3, znh, onesH)


def _edge_body1(xl_h, idx_h, znh_h, ones_h, agg_o, idx_v, rows_v, agg_s, sem):
    return _edge_body(xl_h, idx_h, znh_h, ones_h, agg_o, None,
                      idx_v, rows_v, agg_s, sem, want_cnt=False)


def _edge_pass1(xl, idx3, znh, onesH):
    mesh = plsc.VectorSubcoreMesh(core_axis_name="c", subcore_axis_name="s")
    k = pl.kernel(
        _edge_body1,
        out_type=jax.ShapeDtypeStruct((NCORES, NACC, H), jnp.float32),
        mesh=mesh,
        scratch_types=[
            pltpu.VMEM((2, CHUNK), jnp.int32),
            pltpu.VMEM((CHUNK, H), jnp.float32),
            pltpu.VMEM_SHARED((NACC, H), jnp.float32),
            pltpu.SemaphoreType.DMA,
        ],
    )
    return k(xl, idx3, znh, onesH)


RB = 1000
RBF = 400


def _pre_body(x_ref, wl_ref, wr_ref, b_ref, xl_ref, xr_ref):
    x = x_ref[...]
    xl_ref[...] = jnp.dot(x, wl_ref[...], preferred_element_type=jnp.float32)
    xr_ref[...] = jnp.dot(x, wr_ref[...], preferred_element_type=jnp.float32) + b_ref[...]


def _pre(x, wl, wr, b):
    return pl.pallas_call(
        _pre_body,
        grid=(N // RB,),
        in_specs=[
            pl.BlockSpec((RB, D), lambda i: (i, 0)),
            pl.BlockSpec((D, H), lambda i: (0, 0)),
            pl.BlockSpec((D, H), lambda i: (0, 0)),
            pl.BlockSpec((1, H), lambda i: (0, 0)),
        ],
        out_specs=[
            pl.BlockSpec((RB, H), lambda i: (i, 0)),
            pl.BlockSpec((RB, H), lambda i: (i, 0)),
        ],
        out_shape=[
            jax.ShapeDtypeStruct((N, H), jnp.float32),
            jax.ShapeDtypeStruct((N, H), jnp.float32),
        ],
    )(x, wl, wr, b)


def _mid_body(a0_ref, a1_ref, c0_ref, c1_ref, xr_ref, wl_ref, wr_ref, b_ref,
              xl1_ref, xr1_ref):
    cnt = jnp.maximum(c0_ref[...][:, 0:1] + c1_ref[...][:, 0:1], 1.0)
    h = jnp.maximum((a0_ref[...] + a1_ref[...]) / cnt + xr_ref[...], 0.0)
    xl1_ref[...] = jnp.dot(h, wl_ref[...], preferred_element_type=jnp.float32)
    xr1_ref[...] = jnp.dot(h, wr_ref[...], preferred_element_type=jnp.float32) + b_ref[...]


def _mid(a0, a1, c0, c1, xr, wl, wr, b):
    return pl.pallas_call(
        _mid_body,
        grid=(N // RB,),
        in_specs=[
            pl.BlockSpec((RB, H), lambda i: (i, 0)),
            pl.BlockSpec((RB, H), lambda i: (i, 0)),
            pl.BlockSpec((RB, H), lambda i: (i, 0)),
            pl.BlockSpec((RB, H), lambda i: (i, 0)),
            pl.BlockSpec((RB, H), lambda i: (i, 0)),
            pl.BlockSpec((H, H), lambda i: (0, 0)),
            pl.BlockSpec((H, H), lambda i: (0, 0)),
            pl.BlockSpec((1, H), lambda i: (0, 0)),
        ],
        out_specs=[
            pl.BlockSpec((RB, H), lambda i: (i, 0)),
            pl.BlockSpec((RB, H), lambda i: (i, 0)),
        ],
        out_shape=[
            jax.ShapeDtypeStruct((N, H), jnp.float32),
            jax.ShapeDtypeStruct((N, H), jnp.float32),
        ],
    )(a0, a1, c0, c1, xr, wl, wr, b)


def _fin_body(a0_ref, a1_ref, c0_ref, c1_ref, xr_ref, batch_ref, root_ref,
              wrl_ref, wpl_ref, blin_ref, out_ref, pooled_s, roots_s, csum_s):
    i = pl.program_id(0)

    @pl.when(i == 0)
    def _():
        pooled_s[...] = jnp.zeros_like(pooled_s)
        roots_s[...] = jnp.zeros_like(roots_s)
        csum_s[...] = jnp.zeros_like(csum_s)

    cnt = jnp.maximum(c0_ref[...][:, 0:1] + c1_ref[...][:, 0:1], 1.0)
    h = jnp.maximum((a0_ref[...] + a1_ref[...]) / cnt + xr_ref[...], 0.0)
    bv = batch_ref[...][:, 0]
    gids = lax.broadcasted_iota(jnp.int32, (B, RBF), 0)
    msel = (bv[None, :] == gids).astype(jnp.float32)
    rowids = i * RBF + lax.broadcasted_iota(jnp.int32, (B, RBF), 1)
    rsel = (root_ref[...][:, 0:1] == rowids).astype(jnp.float32)
    pooled_s[...] += jnp.dot(msel, h, preferred_element_type=jnp.float32)
    roots_s[...] += jnp.dot(rsel, h, preferred_element_type=jnp.float32)
    csum_s[...] = csum_s[...] + jnp.sum(msel, axis=1, keepdims=True)

    @pl.when(i == pl.num_programs(0) - 1)
    def _():
        pooled = pooled_s[...] / jnp.maximum(csum_s[...], 1.0)
        logits = (jnp.dot(roots_s[...], wrl_ref[...], preferred_element_type=jnp.float32)
                  + jnp.dot(pooled, wpl_ref[...], preferred_element_type=jnp.float32)
                  + blin_ref[...])
        m = jnp.max(logits, axis=-1, keepdims=True)
        e = logits - m
        lse = jnp.log(jnp.sum(jnp.exp(e), axis=-1, keepdims=True))
        out_ref[...] = e - lse


def _fin(a0, a1, c0, c1, xr, batch8, root8, wrl, wpl, blin):
    return pl.pallas_call(
        _fin_body,
        grid=(N // RBF,),
        in_specs=[
            pl.BlockSpec((RBF, H), lambda i: (i, 0)),
            pl.BlockSpec((RBF, H), lambda i: (i, 0)),
            pl.BlockSpec((RBF, H), lambda i: (i, 0)),
            pl.BlockSpec((RBF, H), lambda i: (i, 0)),
            pl.BlockSpec((RBF, H), lambda i: (i, 0)),
            pl.BlockSpec((RBF, 8), lambda i: (i, 0)),
            pl.BlockSpec((B, 8), lambda i: (0, 0)),
            pl.BlockSpec((H, OUT), lambda i: (0, 0)),
            pl.BlockSpec((H, OUT), lambda i: (0, 0)),
            pl.BlockSpec((1, OUT), lambda i: (0, 0)),
        ],
        out_specs=pl.BlockSpec((B, OUT), lambda i: (0, 0)),
        out_shape=jax.ShapeDtypeStruct((B, OUT), jnp.float32),
        scratch_shapes=[
            pltpu.VMEM((B, H), jnp.float32),
            pltpu.VMEM((B, H), jnp.float32),
            pltpu.VMEM((B, H), jnp.float32),
        ],
    )(a0, a1, c0, c1, xr, batch8, root8, wrl, wpl, blin)


def kernel(x, edge_index, batch, root_n_id, Wl0, Wr0, b0, Wl1, Wr1, b1, Wlin, blin):
    E = edge_index.shape[1]
    ei = edge_index.astype(jnp.int32)
    cpt = -(-E // (NTILES * CHUNK))
    epad = NTILES * cpt * CHUNK
    src3 = jnp.concatenate(
        [ei[0], jnp.zeros((epad - E,), jnp.int32)]).reshape(NTILES, cpt, 1, CHUNK)
    dst3 = jnp.concatenate(
        [ei[1], jnp.full((epad - E,), DUMMY, jnp.int32)]).reshape(NTILES, cpt, 1, CHUNK)
    idx3 = jnp.concatenate([src3, dst3], axis=2)
    znh = jnp.zeros((NACC, H), jnp.float32)
    onesH = jnp.ones((CHUNK, H), jnp.float32)
    batch8 = jnp.broadcast_to(batch.astype(jnp.int32)[:, None], (N, 8))
    root8 = jnp.broadcast_to(root_n_id.astype(jnp.int32)[:, None], (B, 8))
    b0r = b0.reshape(1, H)
    b1r = b1.reshape(1, H)
    blinr = blin.reshape(1, OUT)
    wrl = Wlin[:H]
    wpl = Wlin[H:]

    xl0, xr0 = _pre(x, Wl0, Wr0, b0r)
    aggp0, cntp = _edge_pass0(xl0, idx3, znh, onesH)
    xl1, xr1 = _mid(aggp0[0, :N], aggp0[1, :N], cntp[0, :N], cntp[1, :N],
                    xr0, Wl1, Wr1, b1r)
    aggp1 = _edge_pass1(xl1, idx3, znh, onesH)
    return _fin(aggp1[0, :N], aggp1[1, :N], cntp[0, :N], cntp[1, :N],
                xr1, batch8, root8, wrl, wpl, blinr)


# speedup vs baseline: 1.7474x; 1.2956x over previous
import functools

import jax
import jax.numpy as jnp
from jax import lax
from jax.experimental import pallas as pl
from jax.experimental.pallas import tpu as pltpu
from jax.experimental.pallas import tpu_sc as plsc

N = 10000
D = 128
H = 128
OUT = 64
B = 512

NCORES = 2
NSUB = 16
NTILES = NCORES * NSUB
CHUNK = 128
NACC = 10112
RPT = NACC // NSUB
DUMMY = N
CW = 16


def _edge_body(xl_h, idx_h, znh_h, ones_h, agg_o, cnt_o,
               idx_v, rows_v, agg_s, sem, want_cnt):
    cid = lax.axis_index("c")
    sid = lax.axis_index("s")
    wid = cid * NSUB + sid
    r0 = sid * RPT
    L = idx_h.shape[1]
    pltpu.sync_copy(znh_h.at[pl.ds(r0, RPT)], agg_s.at[pl.ds(r0, RPT)])
    plsc.subcore_barrier()

    def body(j, carry):
        pltpu.sync_copy(idx_h.at[wid, j], idx_v)
        pltpu.async_copy(xl_h.at[idx_v.at[0]], rows_v, sem).wait()
        pltpu.sync_copy(rows_v, agg_s.at[idx_v.at[1]], add=True)
        return carry

    lax.fori_loop(0, L, body, 0)
    plsc.subcore_barrier()
    pltpu.sync_copy(agg_s.at[pl.ds(r0, RPT)], agg_o.at[cid, pl.ds(r0, RPT)])

    if want_cnt:
        plsc.subcore_barrier()
        pltpu.sync_copy(znh_h.at[pl.ds(r0, RPT)], agg_s.at[pl.ds(r0, RPT)])
        pltpu.sync_copy(ones_h, rows_v)
        plsc.subcore_barrier()

        def cbody(j, carry):
            pltpu.sync_copy(idx_h.at[wid, j], idx_v)
            pltpu.sync_copy(rows_v, agg_s.at[idx_v.at[1]], add=True)
            return carry

        lax.fori_loop(0, L, cbody, 0)
        plsc.subcore_barrier()
        pltpu.sync_copy(agg_s.at[pl.ds(r0, RPT)], cnt_o.at[cid, pl.ds(r0, RPT)])


def _edge_pass0(xl, idx3, znh, onesH):
    mesh = plsc.VectorSubcoreMesh(core_axis_name="c", subcore_axis_name="s")
    k = pl.kernel(
        functools.partial(_edge_body, want_cnt=True),
        out_type=[
            jax.ShapeDtypeStruct((NCORES, NACC, H), jnp.float32),
            jax.ShapeDtypeStruct((NCORES, NACC, H), jnp.float32),
        ],
        mesh=mesh,
        scratch_types=[
            pltpu.VMEM((2, CHUNK), jnp.int32),
            pltpu.VMEM((CHUNK, H), jnp.float32),
            pltpu.VMEM_SHARED((NACC, H), jnp.float32),
            pltpu.SemaphoreType.DMA,
        ],
    )
    return k(xl, idx3, znh, onesH)


def _edge_body1(xl_h, idx_h, znh_h, ones_h, agg_o, idx_v, rows_v, agg_s, sem):
    return _edge_body(xl_h, idx_h, znh_h, ones_h, agg_o, None,
                      idx_v, rows_v, agg_s, sem, want_cnt=False)


def _edge_pass1(xl, idx3, znh, onesH):
    mesh = plsc.VectorSubcoreMesh(core_axis_name="c", subcore_axis_name="s")
    k = pl.kernel(
        _edge_body1,
        out_type=jax.ShapeDtypeStruct((NCORES, NACC, H), jnp.float32),
        mesh=mesh,
        scratch_types=[
            pltpu.VMEM((2, CHUNK), jnp.int32),
            pltpu.VMEM((CHUNK, H), jnp.float32),
            pltpu.VMEM_SHARED((NACC, H), jnp.float32),
            pltpu.SemaphoreType.DMA,
        ],
    )
    return k(xl, idx3, znh, onesH)


BACC = 640
RPTB = BACC // NSUB
DUMMYB = B
KP = 3
NROOT = 1024


def _pool_body(h2_h, nidx_h, root_h, znh_h, ones_h, pool_o, cnt_o, root_o,
               idx_v, rows_v, ridx_v, rrows_v, agg_s, sem):
    cid = lax.axis_index("c")
    sid = lax.axis_index("s")
    wid = cid * NSUB + sid
    r0 = sid * RPTB
    pltpu.sync_copy(znh_h.at[pl.ds(r0, RPTB)], agg_s.at[pl.ds(r0, RPTB)])
    plsc.subcore_barrier()

    def body(j, carry):
        pltpu.sync_copy(nidx_h.at[wid, j], idx_v)
        pltpu.async_copy(h2_h.at[idx_v.at[0]], rows_v, sem).wait()
        pltpu.sync_copy(rows_v, agg_s.at[idx_v.at[1]], add=True)
        return carry

    lax.fori_loop(0, KP, body, 0)
    plsc.subcore_barrier()
    pltpu.sync_copy(agg_s.at[pl.ds(r0, RPTB)], pool_o.at[cid, pl.ds(r0, RPTB)])

    plsc.subcore_barrier()
    pltpu.sync_copy(znh_h.at[pl.ds(r0, RPTB)], agg_s.at[pl.ds(r0, RPTB)])
    pltpu.sync_copy(ones_h, rows_v)
    plsc.subcore_barrier()

    def cbody(j, carry):
        pltpu.sync_copy(nidx_h.at[wid, j], idx_v)
        pltpu.sync_copy(rows_v, agg_s.at[idx_v.at[1]], add=True)
        return carry

    lax.fori_loop(0, KP, cbody, 0)
    plsc.subcore_barrier()
    pltpu.sync_copy(agg_s.at[pl.ds(r0, RPTB)], cnt_o.at[cid, pl.ds(r0, RPTB)])

    pltpu.sync_copy(root_h.at[wid], ridx_v.at[0])
    pltpu.async_copy(h2_h.at[ridx_v.at[0]], rrows_v, sem).wait()
    pltpu.sync_copy(rrows_v, root_o.at[pl.ds(pl.multiple_of(wid * 32, 32), 32)])


def _pool_pass(h2, nidx3, rootpad, znh, onesH):
    mesh = plsc.VectorSubcoreMesh(core_axis_name="c", subcore_axis_name="s")
    k = pl.kernel(
        _pool_body,
        out_type=[
            jax.ShapeDtypeStruct((NCORES, BACC, H), jnp.float32),
            jax.ShapeDtypeStruct((NCORES, BACC, H), jnp.float32),
            jax.ShapeDtypeStruct((NROOT, H), jnp.float32),
        ],
        mesh=mesh,
        scratch_types=[
            pltpu.VMEM((2, CHUNK), jnp.int32),
            pltpu.VMEM((CHUNK, H), jnp.float32),
            pltpu.VMEM((1, 32), jnp.int32),
            pltpu.VMEM((32, H), jnp.float32),
            pltpu.VMEM_SHARED((BACC, H), jnp.float32),
            pltpu.SemaphoreType.DMA,
        ],
    )
    return k(h2, nidx3, rootpad, znh, onesH)


RB = 1000
RBF = 400


def _pre_body(x_ref, wl_ref, wr_ref, b_ref, xl_ref, xr_ref):
    x = x_ref[...]
    xl_ref[...] = jnp.dot(x, wl_ref[...], preferred_element_type=jnp.float32)
    xr_ref[...] = jnp.dot(x, wr_ref[...], preferred_element_type=jnp.float32) + b_ref[...]


def _pre(x, wl, wr, b):
    return pl.pallas_call(
        _pre_body,
        grid=(N // RB,),
        in_specs=[
            pl.BlockSpec((RB, D), lambda i: (i, 0)),
            pl.BlockSpec((D, H), lambda i: (0, 0)),
            pl.BlockSpec((D, H), lambda i: (0, 0)),
            pl.BlockSpec((1, H), lambda i: (0, 0)),
        ],
        out_specs=[
            pl.BlockSpec((RB, H), lambda i: (i, 0)),
            pl.BlockSpec((RB, H), lambda i: (i, 0)),
        ],
        out_shape=[
            jax.ShapeDtypeStruct((N, H), jnp.float32),
            jax.ShapeDtypeStruct((N, H), jnp.float32),
        ],
    )(x, wl, wr, b)


def _mid_body(a0_ref, a1_ref, c0_ref, c1_ref, xr_ref, wl_ref, wr_ref, b_ref,
              xl1_ref, xr1_ref):
    cnt = jnp.maximum(c0_ref[...][:, 0:1] + c1_ref[...][:, 0:1], 1.0)
    h = jnp.maximum((a0_ref[...] + a1_ref[...]) / cnt + xr_ref[...], 0.0)
    xl1_ref[...] = jnp.dot(h, wl_ref[...], preferred_element_type=jnp.float32)
    xr1_ref[...] = jnp.dot(h, wr_ref[...], preferred_element_type=jnp.float32) + b_ref[...]


def _mid(a0, a1, c0, c1, xr, wl, wr, b):
    return pl.pallas_call(
        _mid_body,
        grid=(N // RB,),
        in_specs=[
            pl.BlockSpec((RB, H), lambda i: (i, 0)),
            pl.BlockSpec((RB, H), lambda i: (i, 0)),
            pl.BlockSpec((RB, H), lambda i: (i, 0)),
            pl.BlockSpec((RB, H), lambda i: (i, 0)),
            pl.BlockSpec((RB, H), lambda i: (i, 0)),
            pl.BlockSpec((H, H), lambda i: (0, 0)),
            pl.BlockSpec((H, H), lambda i: (0, 0)),
            pl.BlockSpec((1, H), lambda i: (0, 0)),
        ],
        out_specs=[
            pl.BlockSpec((RB, H), lambda i: (i, 0)),
            pl.BlockSpec((RB, H), lambda i: (i, 0)),
        ],
        out_shape=[
            jax.ShapeDtypeStruct((N, H), jnp.float32),
            jax.ShapeDtypeStruct((N, H), jnp.float32),
        ],
    )(a0, a1, c0, c1, xr, wl, wr, b)


def _post_body(a0_ref, a1_ref, c0_ref, c1_ref, xr_ref, h2_ref):
    cnt = jnp.maximum(c0_ref[...][:, 0:1] + c1_ref[...][:, 0:1], 1.0)
    h2_ref[...] = jnp.maximum((a0_ref[...] + a1_ref[...]) / cnt + xr_ref[...], 0.0)


def _post(a0, a1, c0, c1, xr):
    return pl.pallas_call(
        _post_body,
        grid=(N // RB,),
        in_specs=[pl.BlockSpec((RB, H), lambda i: (i, 0))] * 5,
        out_specs=pl.BlockSpec((RB, H), lambda i: (i, 0)),
        out_shape=jax.ShapeDtypeStruct((N, H), jnp.float32),
    )(a0, a1, c0, c1, xr)


def _fin2_body(p0_ref, p1_ref, c0_ref, c1_ref, root_ref, wrl_ref, wpl_ref,
               blin_ref, out_ref):
    cnt = jnp.maximum(c0_ref[...][:, 0:1] + c1_ref[...][:, 0:1], 1.0)
    pooled = (p0_ref[...] + p1_ref[...]) / cnt
    logits = (jnp.dot(root_ref[...], wrl_ref[...], preferred_element_type=jnp.float32)
              + jnp.dot(pooled, wpl_ref[...], preferred_element_type=jnp.float32)
              + blin_ref[...])
    m = jnp.max(logits, axis=-1, keepdims=True)
    e = logits - m
    lse = jnp.log(jnp.sum(jnp.exp(e), axis=-1, keepdims=True))
    out_ref[...] = e - lse


def _fin2(p0, p1, c0, c1, root, wrl, wpl, blin):
    return pl.pallas_call(
        _fin2_body,
        grid=(1,),
        in_specs=[
            pl.BlockSpec((B, H), lambda i: (0, 0)),
            pl.BlockSpec((B, H), lambda i: (0, 0)),
            pl.BlockSpec((B, H), lambda i: (0, 0)),
            pl.BlockSpec((B, H), lambda i: (0, 0)),
            pl.BlockSpec((B, H), lambda i: (0, 0)),
            pl.BlockSpec((H, OUT), lambda i: (0, 0)),
            pl.BlockSpec((H, OUT), lambda i: (0, 0)),
            pl.BlockSpec((1, OUT), lambda i: (0, 0)),
        ],
        out_specs=pl.BlockSpec((B, OUT), lambda i: (0, 0)),
        out_shape=jax.ShapeDtypeStruct((B, OUT), jnp.float32),
    )(p0, p1, c0, c1, root, wrl, wpl, blin)


def kernel(x, edge_index, batch, root_n_id, Wl0, Wr0, b0, Wl1, Wr1, b1, Wlin, blin):
    E = edge_index.shape[1]
    ei = edge_index.astype(jnp.int32)
    cpt = -(-E // (NTILES * CHUNK))
    epad = NTILES * cpt * CHUNK
    src3 = jnp.concatenate(
        [ei[0], jnp.zeros((epad - E,), jnp.int32)]).reshape(NTILES, cpt, 1, CHUNK)
    dst3 = jnp.concatenate(
        [ei[1], jnp.full((epad - E,), DUMMY, jnp.int32)]).reshape(NTILES, cpt, 1, CHUNK)
    idx3 = jnp.concatenate([src3, dst3], axis=2)
    npad = NTILES * KP * CHUNK
    nsrc = jnp.concatenate(
        [jnp.arange(N, dtype=jnp.int32),
         jnp.zeros((npad - N,), jnp.int32)]).reshape(NTILES, KP, 1, CHUNK)
    ndst = jnp.concatenate(
        [batch.astype(jnp.int32),
         jnp.full((npad - N,), DUMMYB, jnp.int32)]).reshape(NTILES, KP, 1, CHUNK)
    nidx3 = jnp.concatenate([nsrc, ndst], axis=2)
    rootpad = jnp.concatenate(
        [root_n_id.astype(jnp.int32),
         jnp.zeros((NROOT - B,), jnp.int32)]).reshape(NTILES, 32)
    znh = jnp.zeros((NACC, H), jnp.float32)
    onesH = jnp.ones((CHUNK, H), jnp.float32)
    b0r = b0.reshape(1, H)
    b1r = b1.reshape(1, H)
    blinr = blin.reshape(1, OUT)
    wrl = Wlin[:H]
    wpl = Wlin[H:]

    xl0, xr0 = _pre(x, Wl0, Wr0, b0r)
    aggp0, cntp = _edge_pass0(xl0, idx3, znh, onesH)
    xl1, xr1 = _mid(aggp0[0, :N], aggp0[1, :N], cntp[0, :N], cntp[1, :N],
                    xr0, Wl1, Wr1, b1r)
    aggp1 = _edge_pass1(xl1, idx3, znh, onesH)
    h2 = _post(aggp1[0, :N], aggp1[1, :N], cntp[0, :N], cntp[1, :N], xr1)
    poolp, pcntp, roots = _pool_pass(h2, nidx3, rootpad, znh, onesH)
    return _fin2(poolp[0, :B], poolp[1, :B], pcntp[0, :B], pcntp[1, :B],
                 roots[:B], wrl, wpl, blinr)


# speedup vs baseline: 2.2677x; 1.2978x over previous
import functools

import jax
import jax.numpy as jnp
from jax import lax
from jax.experimental import pallas as pl
from jax.experimental.pallas import tpu as pltpu
from jax.experimental.pallas import tpu_sc as plsc

N = 10000
D = 128
H = 128
OUT = 64
B = 512

NCORES = 2
NSUB = 16
NTILES = NCORES * NSUB
CHUNK = 128
NACC = 10112
RPT = NACC // NSUB
DUMMY = N
K0 = 90
K1 = 67
TOTC = NSUB * (K0 + K1)


def _edge_body(xl_h, idx_h, znh_h, ones_h, agg_o, cnt_o,
               idx_v, rows_v, agg_s, sem, want_cnt):
    cid = lax.axis_index("c")
    sid = lax.axis_index("s")
    wid = cid * NSUB + sid
    r0 = sid * RPT
    base = lax.select(cid == 0, sid * K0, NSUB * K0 + sid * K1)
    Lt = lax.select(cid == 0, K0, K1)
    pltpu.sync_copy(znh_h.at[pl.ds(r0, RPT)], agg_s.at[pl.ds(r0, RPT)])
    plsc.subcore_barrier()

    def body(j, carry):
        pltpu.sync_copy(idx_h.at[base + j], idx_v)
        pltpu.async_copy(xl_h.at[idx_v.at[0]], rows_v, sem).wait()
        pltpu.sync_copy(rows_v, agg_s.at[idx_v.at[1]], add=True)
        return carry

    lax.fori_loop(0, Lt, body, 0)
    plsc.subcore_barrier()
    pltpu.sync_copy(agg_s.at[pl.ds(r0, RPT)], agg_o.at[cid, pl.ds(r0, RPT)])

    if want_cnt:
        plsc.subcore_barrier()
        pltpu.sync_copy(znh_h.at[pl.ds(r0, RPT)], agg_s.at[pl.ds(r0, RPT)])
        pltpu.sync_copy(ones_h, rows_v)
        plsc.subcore_barrier()

        def cbody(j, carry):
            pltpu.sync_copy(idx_h.at[base + j], idx_v)
            pltpu.sync_copy(rows_v, agg_s.at[idx_v.at[1]], add=True)
            return carry

        lax.fori_loop(0, Lt, cbody, 0)
        plsc.subcore_barrier()
        pltpu.sync_copy(agg_s.at[pl.ds(r0, RPT)], cnt_o.at[cid, pl.ds(r0, RPT)])


def _edge_pass0(xl, idx3, znh, onesH):
    mesh = plsc.VectorSubcoreMesh(core_axis_name="c", subcore_axis_name="s")
    k = pl.kernel(
        functools.partial(_edge_body, want_cnt=True),
        out_type=[
            jax.ShapeDtypeStruct((NCORES, NACC, H), jnp.float32),
            jax.ShapeDtypeStruct((NCORES, NACC, H), jnp.float32),
        ],
        mesh=mesh,
        scratch_types=[
            pltpu.VMEM((2, CHUNK), jnp.int32),
            pltpu.VMEM((CHUNK, H), jnp.float32),
            pltpu.VMEM_SHARED((NACC, H), jnp.float32),
            pltpu.SemaphoreType.DMA,
        ],
    )
    return k(xl, idx3, znh, onesH)


def _edge_body1(xl_h, idx_h, znh_h, ones_h, agg_o, idx_v, rows_v, agg_s, sem):
    return _edge_body(xl_h, idx_h, znh_h, ones_h, agg_o, None,
                      idx_v, rows_v, agg_s, sem, want_cnt=False)


def _edge_pass1(xl, idx3, znh, onesH):
    mesh = plsc.VectorSubcoreMesh(core_axis_name="c", subcore_axis_name="s")
    k = pl.kernel(
        _edge_body1,
        out_type=jax.ShapeDtypeStruct((NCORES, NACC, H), jnp.float32),
        mesh=mesh,
        scratch_types=[
            pltpu.VMEM((2, CHUNK), jnp.int32),
            pltpu.VMEM((CHUNK, H), jnp.float32),
            pltpu.VMEM_SHARED((NACC, H), jnp.float32),
            pltpu.SemaphoreType.DMA,
        ],
    )
    return k(xl, idx3, znh, onesH)


BACC = 640
RPTB = BACC // NSUB
DUMMYB = B
KP = 3
NROOT = 1024


def _pool_body(h2_h, nidx_h, root_h, znh_h, ones_h, pool_o, cnt_o, root_o,
               idx_v, rows_v, ridx_v, rrows_v, agg_s, sem):
    cid = lax.axis_index("c")
    sid = lax.axis_index("s")
    wid = cid * NSUB + sid
    r0 = sid * RPTB
    pltpu.sync_copy(znh_h.at[pl.ds(r0, RPTB)], agg_s.at[pl.ds(r0, RPTB)])
    plsc.subcore_barrier()

    def body(j, carry):
        pltpu.sync_copy(nidx_h.at[wid, j], idx_v)
        pltpu.async_copy(h2_h.at[idx_v.at[0]], rows_v, sem).wait()
        pltpu.sync_copy(rows_v, agg_s.at[idx_v.at[1]], add=True)
        return carry

    lax.fori_loop(0, KP, body, 0)
    plsc.subcore_barrier()
    pltpu.sync_copy(agg_s.at[pl.ds(r0, RPTB)], pool_o.at[cid, pl.ds(r0, RPTB)])

    plsc.subcore_barrier()
    pltpu.sync_copy(znh_h.at[pl.ds(r0, RPTB)], agg_s.at[pl.ds(r0, RPTB)])
    pltpu.sync_copy(ones_h, rows_v)
    plsc.subcore_barrier()

    def cbody(j, carry):
        pltpu.sync_copy(nidx_h.at[wid, j], idx_v)
        pltpu.sync_copy(rows_v, agg_s.at[idx_v.at[1]], add=True)
        return carry

    lax.fori_loop(0, KP, cbody, 0)
    plsc.subcore_barrier()
    pltpu.sync_copy(agg_s.at[pl.ds(r0, RPTB)], cnt_o.at[cid, pl.ds(r0, RPTB)])

    pltpu.sync_copy(root_h.at[wid], ridx_v.at[0])
    pltpu.async_copy(h2_h.at[ridx_v.at[0]], rrows_v, sem).wait()
    pltpu.sync_copy(rrows_v, root_o.at[pl.ds(pl.multiple_of(wid * 32, 32), 32)])


def _pool_pass(h2, nidx3, rootpad, znh, onesH):
    mesh = plsc.VectorSubcoreMesh(core_axis_name="c", subcore_axis_name="s")
    k = pl.kernel(
        _pool_body,
        out_type=[
            jax.ShapeDtypeStruct((NCORES, BACC, H), jnp.float32),
            jax.ShapeDtypeStruct((NCORES, BACC, H), jnp.float32),
            jax.ShapeDtypeStruct((NROOT, H), jnp.float32),
        ],
        mesh=mesh,
        scratch_types=[
            pltpu.VMEM((2, CHUNK), jnp.int32),
            pltpu.VMEM((CHUNK, H), jnp.float32),
            pltpu.VMEM((1, 32), jnp.int32),
            pltpu.VMEM((32, H), jnp.float32),
            pltpu.VMEM_SHARED((BACC, H), jnp.float32),
            pltpu.SemaphoreType.DMA,
        ],
    )
    return k(h2, nidx3, rootpad, znh, onesH)


RB = 1000
RBF = 400


def _pre_body(x_ref, wl_ref, wr_ref, b_ref, xl_ref, xr_ref):
    x = x_ref[...]
    xl_ref[...] = jnp.dot(x, wl_ref[...], preferred_element_type=jnp.float32)
    xr_ref[...] = jnp.dot(x, wr_ref[...], preferred_element_type=jnp.float32) + b_ref[...]


def _pre(x, wl, wr, b):
    return pl.pallas_call(
        _pre_body,
        grid=(N // RB,),
        in_specs=[
            pl.BlockSpec((RB, D), lambda i: (i, 0)),
            pl.BlockSpec((D, H), lambda i: (0, 0)),
            pl.BlockSpec((D, H), lambda i: (0, 0)),
            pl.BlockSpec((1, H), lambda i: (0, 0)),
        ],
        out_specs=[
            pl.BlockSpec((RB, H), lambda i: (i, 0)),
            pl.BlockSpec((RB, H), lambda i: (i, 0)),
        ],
        out_shape=[
            jax.ShapeDtypeStruct((N, H), jnp.float32),
            jax.ShapeDtypeStruct((N, H), jnp.float32),
        ],
    )(x, wl, wr, b)


def _mid_body(a0_ref, a1_ref, c0_ref, c1_ref, xr_ref, wl_ref, wr_ref, b_ref,
              xl1_ref, xr1_ref):
    cnt = jnp.maximum(c0_ref[...][:, 0:1] + c1_ref[...][:, 0:1], 1.0)
    h = jnp.maximum((a0_ref[...] + a1_ref[...]) / cnt + xr_ref[...], 0.0)
    xl1_ref[...] = jnp.dot(h, wl_ref[...], preferred_element_type=jnp.float32)
    xr1_ref[...] = jnp.dot(h, wr_ref[...], preferred_element_type=jnp.float32) + b_ref[...]


def _mid(a0, a1, c0, c1, xr, wl, wr, b):
    return pl.pallas_call(
        _mid_body,
        grid=(N // RB,),
        in_specs=[
            pl.BlockSpec((RB, H), lambda i: (i, 0)),
            pl.BlockSpec((RB, H), lambda i: (i, 0)),
            pl.BlockSpec((RB, H), lambda i: (i, 0)),
            pl.BlockSpec((RB, H), lambda i: (i, 0)),
            pl.BlockSpec((RB, H), lambda i: (i, 0)),
            pl.BlockSpec((H, H), lambda i: (0, 0)),
            pl.BlockSpec((H, H), lambda i: (0, 0)),
            pl.BlockSpec((1, H), lambda i: (0, 0)),
        ],
        out_specs=[
            pl.BlockSpec((RB, H), lambda i: (i, 0)),
            pl.BlockSpec((RB, H), lambda i: (i, 0)),
        ],
        out_shape=[
            jax.ShapeDtypeStruct((N, H), jnp.float32),
            jax.ShapeDtypeStruct((N, H), jnp.float32),
        ],
    )(a0, a1, c0, c1, xr, wl, wr, b)


def _post_body(a0_ref, a1_ref, c0_ref, c1_ref, xr_ref, h2_ref):
    cnt = jnp.maximum(c0_ref[...][:, 0:1] + c1_ref[...][:, 0:1], 1.0)
    h2_ref[...] = jnp.maximum((a0_ref[...] + a1_ref[...]) / cnt + xr_ref[...], 0.0)


def _post(a0, a1, c0, c1, xr):
    return pl.pallas_call(
        _post_body,
        grid=(N // RB,),
        in_specs=[pl.BlockSpec((RB, H), lambda i: (i, 0))] * 5,
        out_specs=pl.BlockSpec((RB, H), lambda i: (i, 0)),
        out_shape=jax.ShapeDtypeStruct((N, H), jnp.float32),
    )(a0, a1, c0, c1, xr)


def _fin2_body(p0_ref, p1_ref, c0_ref, c1_ref, root_ref, wrl_ref, wpl_ref,
               blin_ref, out_ref):
    cnt = jnp.maximum(c0_ref[...][:, 0:1] + c1_ref[...][:, 0:1], 1.0)
    pooled = (p0_ref[...] + p1_ref[...]) / cnt
    logits = (jnp.dot(root_ref[...], wrl_ref[...], preferred_element_type=jnp.float32)
              + jnp.dot(pooled, wpl_ref[...], preferred_element_type=jnp.float32)
              + blin_ref[...])
    m = jnp.max(logits, axis=-1, keepdims=True)
    e = logits - m
    lse = jnp.log(jnp.sum(jnp.exp(e), axis=-1, keepdims=True))
    out_ref[...] = e - lse


def _fin2(p0, p1, c0, c1, root, wrl, wpl, blin):
    return pl.pallas_call(
        _fin2_body,
        grid=(1,),
        in_specs=[
            pl.BlockSpec((B, H), lambda i: (0, 0)),
            pl.BlockSpec((B, H), lambda i: (0, 0)),
            pl.BlockSpec((B, H), lambda i: (0, 0)),
            pl.BlockSpec((B, H), lambda i: (0, 0)),
            pl.BlockSpec((B, H), lambda i: (0, 0)),
            pl.BlockSpec((H, OUT), lambda i: (0, 0)),
            pl.BlockSpec((H, OUT), lambda i: (0, 0)),
            pl.BlockSpec((1, OUT), lambda i: (0, 0)),
        ],
        out_specs=pl.BlockSpec((B, OUT), lambda i: (0, 0)),
        out_shape=jax.ShapeDtypeStruct((B, OUT), jnp.float32),
    )(p0, p1, c0, c1, root, wrl, wpl, blin)


def kernel(x, edge_index, batch, root_n_id, Wl0, Wr0, b0, Wl1, Wr1, b1, Wlin, blin):
    E = edge_index.shape[1]
    ei = edge_index.astype(jnp.int32)
    epad = TOTC * CHUNK
    src3 = jnp.concatenate(
        [ei[0], jnp.zeros((epad - E,), jnp.int32)]).reshape(TOTC, 1, CHUNK)
    dst3 = jnp.concatenate(
        [ei[1], jnp.full((epad - E,), DUMMY, jnp.int32)]).reshape(TOTC, 1, CHUNK)
    idx3 = jnp.concatenate([src3, dst3], axis=1)
    npad = NTILES * KP * CHUNK
    nsrc = jnp.concatenate(
        [jnp.arange(N, dtype=jnp.int32),
         jnp.zeros((npad - N,), jnp.int32)]).reshape(NTILES, KP, 1, CHUNK)
    ndst = jnp.concatenate(
        [batch.astype(jnp.int32),
         jnp.full((npad - N,), DUMMYB, jnp.int32)]).reshape(NTILES, KP, 1, CHUNK)
    nidx3 = jnp.concatenate([nsrc, ndst], axis=2)
    rootpad = jnp.concatenate(
        [root_n_id.astype(jnp.int32),
         jnp.zeros((NROOT - B,), jnp.int32)]).reshape(NTILES, 32)
    znh = jnp.zeros((NACC, H), jnp.float32)
    onesH = jnp.ones((CHUNK, H), jnp.float32)
    b0r = b0.reshape(1, H)
    b1r = b1.reshape(1, H)
    blinr = blin.reshape(1, OUT)
    wrl = Wlin[:H]
    wpl = Wlin[H:]

    xl0, xr0 = _pre(x, Wl0, Wr0, b0r)
    aggp0, cntp = _edge_pass0(xl0, idx3, znh, onesH)
    xl1, xr1 = _mid(aggp0[0, :N], aggp0[1, :N], cntp[0, :N], cntp[1, :N],
                    xr0, Wl1, Wr1, b1r)
    aggp1 = _edge_pass1(xl1, idx3, znh, onesH)
    h2 = _post(aggp1[0, :N], aggp1[1, :N], cntp[0, :N], cntp[1, :N], xr1)
    poolp, pcntp, roots = _pool_pass(h2, nidx3, rootpad, znh, onesH)
    return _fin2(poolp[0, :B], poolp[1, :B], pcntp[0, :B], pcntp[1, :B],
                 roots[:B], wrl, wpl, blinr)
